# Initial kernel scaffold; baseline (speedup 1.0000x reference)
import jax
import jax.numpy as jnp
from jax.experimental import pallas as pl


def kernel(x, W_bin, W_pos):
    raise NotImplementedError("write your pallas kernel here")


# speedup vs baseline: 5.5552x; 5.5552x over previous
import jax
import jax.numpy as jnp
from jax import lax
from jax.experimental import pallas as pl
from jax.experimental.pallas import tpu as pltpu
from jax.experimental.pallas import tpu_sc as plsc

D = 32
N = 512
B = 4
K = 4
TLEN = 1024
NC, NS, L = 2, 16, 16
NW = NC * NS
ROWS = B * N
RPW = ROWS // NW
TPT = TLEN // NW
WPB = NW // B
TWIN = 576

_mesh = plsc.VectorSubcoreMesh(core_axis_name="c", subcore_axis_name="s")
_params = pltpu.CompilerParams(use_tc_tiling_on_sc=False)


def _build_body(wbin_hbm, wpos_hbm, t_hbm, wpos_v, wbin_v, tbuf):
    cid = lax.axis_index("c")
    sid = lax.axis_index("s")
    wid = cid * NS + sid
    pltpu.sync_copy(wpos_hbm, wpos_v)
    pltpu.sync_copy(wbin_hbm, wbin_v)
    t0 = wid * TPT

    def build_t(tl, carry):
        t = t0 + tl
        widx = jnp.clip(t - 383, 0, 256)
        for h in range(2):
            pv = wpos_v[widx, pl.ds(h * L, L)]
            for k in range(K):
                tbuf[k, tl, pl.ds(h * L, L)] = pv + wbin_v[k, pl.ds(h * L, L)]
        return carry

    lax.fori_loop(0, TPT, build_t, 0)
    for k in range(K):
        pltpu.sync_copy(tbuf.at[k], t_hbm.at[pl.ds(k * TLEN + t0, TPT)])


_sc_build = pl.kernel(
    _build_body,
    out_type=jax.ShapeDtypeStruct((K * TLEN, D), jnp.float32),
    mesh=_mesh,
    scratch_types=[
        pltpu.VMEM((257, D), jnp.float32),
        pltpu.VMEM((K, D), jnp.float32),
        pltpu.VMEM((K, TPT, D), jnp.float32),
    ],
    compiler_params=_params,
)


def _lookup_body(x_hbm, t_hbm, out_hbm, twin, xbuf, rowbuf):
    cid = lax.axis_index("c")
    sid = lax.axis_index("s")
    wid = cid * NS + sid
    b = wid // WPB
    i0 = (wid % WPB) * RPW

    twin0 = 448 - i0
    for k in range(K):
        pltpu.sync_copy(t_hbm.at[pl.ds(k * TLEN + twin0, TWIN)],
                        twin.at[pl.ds(k * TWIN, TWIN)])

    def row_step(rr, carry):
        r = (b * N + i0) + rr
        pltpu.sync_copy(x_hbm.at[r], xbuf)
        toff = 63 - rr

        def pos_step(pp, carry2):
            p0 = pp * L
            xv = xbuf[pl.ds(p0, L)]
            base = xv * TWIN + (p0 + toff)
            for u in range(L):
                row = base[u] + u
                for h in range(2):
                    rowbuf[p0 + u, pl.ds(h * L, L)] = twin[row, pl.ds(h * L, L)]
            return carry2

        lax.fori_loop(0, N // L, pos_step, 0)
        pltpu.sync_copy(rowbuf, out_hbm.at[r])
        return carry

    lax.fori_loop(0, RPW, row_step, 0)


_sc_lookup = pl.kernel(
    _lookup_body,
    out_type=jax.ShapeDtypeStruct((ROWS, N, D), jnp.float32),
    mesh=_mesh,
    scratch_types=[
        pltpu.VMEM((K * TWIN, D), jnp.float32),
        pltpu.VMEM((N,), jnp.int32),
        pltpu.VMEM((N, D), jnp.float32),
    ],
    compiler_params=_params,
)


@jax.jit
def kernel(x, W_bin, W_pos):
    t = _sc_build(W_bin, W_pos)
    x2 = x.astype(jnp.int32).reshape(ROWS, N)
    return _sc_lookup(x2, t).reshape(B, N, N, D)


# speedup vs baseline: 6.0778x; 1.0941x over previous
import jax
import jax.numpy as jnp
from jax import lax
from jax.experimental import pallas as pl
from jax.experimental.pallas import tpu as pltpu
from jax.experimental.pallas import tpu_sc as plsc

D = 32
N = 512
B = 4
K = 4
TLEN = 1024
NC, NS, L = 2, 16, 16
NW = NC * NS
ROWS = B * N
RPW = ROWS // NW
TPT = TLEN // NW
WPB = NW // B
TWIN = 576

_mesh = plsc.VectorSubcoreMesh(core_axis_name="c", subcore_axis_name="s")
_params = pltpu.CompilerParams(use_tc_tiling_on_sc=False)


def _build_body(wbin_hbm, wpos_hbm, t_hbm, wpos_v, wbin_v, tbuf):
    cid = lax.axis_index("c")
    sid = lax.axis_index("s")
    wid = cid * NS + sid
    pltpu.sync_copy(wpos_hbm, wpos_v)
    pltpu.sync_copy(wbin_hbm, wbin_v)
    t0 = wid * TPT

    def build_t(tl, carry):
        t = t0 + tl
        widx = jnp.clip(t - 383, 0, 256)
        for h in range(2):
            pv = wpos_v[widx, pl.ds(h * L, L)]
            for k in range(K):
                tbuf[k, tl, pl.ds(h * L, L)] = pv + wbin_v[k, pl.ds(h * L, L)]
        return carry

    lax.fori_loop(0, TPT, build_t, 0)
    for k in range(K):
        pltpu.sync_copy(tbuf.at[k], t_hbm.at[pl.ds(k * TLEN + t0, TPT)])


_sc_build = pl.kernel(
    _build_body,
    out_type=jax.ShapeDtypeStruct((K * TLEN, D), jnp.float32),
    mesh=_mesh,
    scratch_types=[
        pltpu.VMEM((257, D), jnp.float32),
        pltpu.VMEM((K, D), jnp.float32),
        pltpu.VMEM((K, TPT, D), jnp.float32),
    ],
    compiler_params=_params,
)


def _lookup_body(x_hbm, t_hbm, out_hbm, twin, xb0, xb1, rb0, rb1,
                 sem_x0, sem_x1, sem_o0, sem_o1):
    cid = lax.axis_index("c")
    sid = lax.axis_index("s")
    wid = cid * NS + sid
    b = wid // WPB
    i0 = (wid % WPB) * RPW
    r_base = b * N + i0
    iota = lax.iota(jnp.int32, L)
    cols = (iota, iota + L)
    slots = ((xb0, rb0, sem_x0, sem_o0), (xb1, rb1, sem_x1, sem_o1))

    twin0 = 448 - i0
    for k in range(K):
        pltpu.sync_copy(t_hbm.at[pl.ds(k * TLEN + twin0, TWIN)],
                        twin.at[pl.ds(k * TWIN, TWIN)])

    pltpu.make_async_copy(x_hbm.at[r_base], xb0, sem_x0).start()

    def pair_step(g, carry):
        for s in range(2):
            xb, rb, sem_x, sem_o = slots[s]
            xb_n, _, sem_x_n, _ = slots[1 - s]
            rr = 2 * g + s
            r = r_base + rr
            toff = 63 - rr

            @pl.when(rr < RPW - 1)
            def _prefetch():
                pltpu.make_async_copy(x_hbm.at[r + 1], xb_n, sem_x_n).start()

            pltpu.make_async_copy(x_hbm.at[r], xb, sem_x).wait()

            @pl.when(rr >= 2)
            def _drain():
                pltpu.make_async_copy(rb, out_hbm.at[r], sem_o).wait()

            def pos_step(pp, carry2):
                p0 = pp * L
                xv = xb[pl.ds(p0, L)]
                basev = xv * TWIN + (iota + (p0 + toff))
                for u in range(L):
                    row = basev[u]
                    for h in range(2):
                        rb[p0 + u, pl.ds(h * L, L)] = twin[row, pl.ds(h * L, L)]
                return carry2

            lax.fori_loop(0, N // L, pos_step, 0)
            pltpu.make_async_copy(rb, out_hbm.at[r], sem_o).start()
        return carry

    lax.fori_loop(0, RPW // 2, pair_step, 0)
    for s in range(2):
        xb, rb, sem_x, sem_o = slots[s]
        pltpu.make_async_copy(rb, out_hbm.at[r_base + RPW - 2 + s], sem_o).wait()


_sc_lookup = pl.kernel(
    _lookup_body,
    out_type=jax.ShapeDtypeStruct((ROWS, N, D), jnp.float32),
    mesh=_mesh,
    scratch_types=[
        pltpu.VMEM((K * TWIN, D), jnp.float32),
        pltpu.VMEM((N,), jnp.int32),
        pltpu.VMEM((N,), jnp.int32),
        pltpu.VMEM((N, D), jnp.float32),
        pltpu.VMEM((N, D), jnp.float32),
        pltpu.SemaphoreType.DMA,
        pltpu.SemaphoreType.DMA,
        pltpu.SemaphoreType.DMA,
        pltpu.SemaphoreType.DMA,
    ],
    compiler_params=_params,
)


@jax.jit
def kernel(x, W_bin, W_pos):
    t = _sc_build(W_bin, W_pos)
    x2 = x.astype(jnp.int32).reshape(ROWS, N)
    return _sc_lookup(x2, t).reshape(B, N, N, D)


# speedup vs baseline: 8.2577x; 1.3587x over previous
import jax
import jax.numpy as jnp
from jax import lax
from jax.experimental import pallas as pl
from jax.experimental.pallas import tpu as pltpu
from jax.experimental.pallas import tpu_sc as plsc

D = 32
N = 512
B = 4
K = 4
TLEN = 1024
NC, NS, L = 2, 16, 16
NW = NC * NS
ROWS = B * N
RPW = ROWS // NW
TPT = TLEN // NW
WPB = NW // B
TWIN = 576

_mesh = plsc.VectorSubcoreMesh(core_axis_name="c", subcore_axis_name="s")
_params = pltpu.CompilerParams(use_tc_tiling_on_sc=False)


def _build_body(wbin_hbm, wpos_hbm, t_hbm, wpos_v, wbin_v, tbuf):
    cid = lax.axis_index("c")
    sid = lax.axis_index("s")
    wid = cid * NS + sid
    pltpu.sync_copy(wpos_hbm, wpos_v)
    pltpu.sync_copy(wbin_hbm, wbin_v)
    t0 = wid * TPT

    def build_t(tl, carry):
        t = t0 + tl
        widx = jnp.clip(t - 383, 0, 256)
        for h in range(2):
            pv = wpos_v[widx, pl.ds(h * L, L)]
            for k in range(K):
                tbuf[k, tl, pl.ds(h * L, L)] = pv + wbin_v[k, pl.ds(h * L, L)]
        return carry

    lax.fori_loop(0, TPT, build_t, 0)
    for k in range(K):
        pltpu.sync_copy(tbuf.at[k], t_hbm.at[pl.ds(k * TLEN + t0, TPT)])


_sc_build = pl.kernel(
    _build_body,
    out_type=jax.ShapeDtypeStruct((K * TLEN, D), jnp.float32),
    mesh=_mesh,
    scratch_types=[
        pltpu.VMEM((257, D), jnp.float32),
        pltpu.VMEM((K, D), jnp.float32),
        pltpu.VMEM((K, TPT, D), jnp.float32),
    ],
    compiler_params=_params,
)


def _lookup_body(x_hbm, t_hbm, out_hbm, twin, xb0, xb1, rb0, rb1,
                 sem_x0, sem_x1, sem_o0, sem_o1):
    cid = lax.axis_index("c")
    sid = lax.axis_index("s")
    wid = cid * NS + sid
    b = wid // WPB
    i0 = (wid % WPB) * RPW
    r_base = b * N + i0
    iota = lax.iota(jnp.int32, L)
    cols = (iota, iota + L)
    slots = ((xb0, rb0, sem_x0, sem_o0), (xb1, rb1, sem_x1, sem_o1))

    twin0 = 448 - i0
    for k in range(K):
        pltpu.sync_copy(t_hbm.at[pl.ds(k * TLEN + twin0, TWIN)],
                        twin.at[pl.ds(k * TWIN, TWIN)])

    pltpu.make_async_copy(x_hbm.at[r_base], xb0, sem_x0).start()

    def pair_step(g, carry):
        for s in range(2):
            xb, rb, sem_x, sem_o = slots[s]
            xb_n, _, sem_x_n, _ = slots[1 - s]
            rr = 2 * g + s
            r = r_base + rr
            toff = 63 - rr

            @pl.when(rr < RPW - 1)
            def _prefetch():
                pltpu.make_async_copy(x_hbm.at[r + 1], xb_n, sem_x_n).start()

            pltpu.make_async_copy(x_hbm.at[r], xb, sem_x).wait()

            @pl.when(rr >= 2)
            def _drain():
                pltpu.make_async_copy(rb, out_hbm.at[r], sem_o).wait()

            @plsc.parallel_loop(0, N, step=L)
            def pos_step(p0):
                xv = xb[pl.ds(p0, L)]
                basev = xv * TWIN + (iota + (p0 + toff))
                for u in range(L):
                    row = basev[u]
                    for h in range(2):
                        rb[p0 + u, pl.ds(h * L, L)] = twin[row, pl.ds(h * L, L)]
            pltpu.make_async_copy(rb, out_hbm.at[r], sem_o).start()
        return carry

    lax.fori_loop(0, RPW // 2, pair_step, 0)
    for s in range(2):
        xb, rb, sem_x, sem_o = slots[s]
        pltpu.make_async_copy(rb, out_hbm.at[r_base + RPW - 2 + s], sem_o).wait()


_sc_lookup = pl.kernel(
    _lookup_body,
    out_type=jax.ShapeDtypeStruct((ROWS, N, D), jnp.float32),
    mesh=_mesh,
    scratch_types=[
        pltpu.VMEM((K * TWIN, D), jnp.float32),
        pltpu.VMEM((N,), jnp.int32),
        pltpu.VMEM((N,), jnp.int32),
        pltpu.VMEM((N, D), jnp.float32),
        pltpu.VMEM((N, D), jnp.float32),
        pltpu.SemaphoreType.DMA,
        pltpu.SemaphoreType.DMA,
        pltpu.SemaphoreType.DMA,
        pltpu.SemaphoreType.DMA,
    ],
    compiler_params=_params,
)


@jax.jit
def kernel(x, W_bin, W_pos):
    t = _sc_build(W_bin, W_pos)
    x2 = x.astype(jnp.int32).reshape(ROWS, N)
    return _sc_lookup(x2, t).reshape(B, N, N, D)


# speedup vs baseline: 8.2720x; 1.0017x over previous
import jax
import jax.numpy as jnp
from jax import lax
from jax.experimental import pallas as pl
from jax.experimental.pallas import tpu as pltpu
from jax.experimental.pallas import tpu_sc as plsc

D = 32
N = 512
B = 4
K = 4
TLEN = 1024
NC, NS, L = 2, 16, 16
NW = NC * NS
ROWS = B * N
RPW = ROWS // NW
TPT = TLEN // NW
WPB = NW // B
TWIN = 576

_mesh = plsc.VectorSubcoreMesh(core_axis_name="c", subcore_axis_name="s")
_params = pltpu.CompilerParams(use_tc_tiling_on_sc=False)


def _build_body(wbin_hbm, wpos_hbm, t_hbm, wpos_v, wbin_v, tbuf):
    cid = lax.axis_index("c")
    sid = lax.axis_index("s")
    wid = cid * NS + sid
    pltpu.sync_copy(wpos_hbm, wpos_v)
    pltpu.sync_copy(wbin_hbm, wbin_v)
    t0 = wid * TPT

    def build_t(tl, carry):
        t = t0 + tl
        widx = jnp.clip(t - 383, 0, 256)
        for h in range(2):
            pv = wpos_v[widx, pl.ds(h * L, L)]
            for k in range(K):
                tbuf[k, tl, pl.ds(h * L, L)] = pv + wbin_v[k, pl.ds(h * L, L)]
        return carry

    lax.fori_loop(0, TPT, build_t, 0)
    for k in range(K):
        pltpu.sync_copy(tbuf.at[k], t_hbm.at[pl.ds(k * TLEN + t0, TPT)])


_sc_build = pl.kernel(
    _build_body,
    out_type=jax.ShapeDtypeStruct((K * TLEN, D), jnp.float32),
    mesh=_mesh,
    scratch_types=[
        pltpu.VMEM((257, D), jnp.float32),
        pltpu.VMEM((K, D), jnp.float32),
        pltpu.VMEM((K, TPT, D), jnp.float32),
    ],
    compiler_params=_params,
)


def _lookup_body(x_hbm, t_hbm, out_hbm, twin, xb0, xb1, rb0, rb1,
                 sem_x0, sem_x1, sem_o0, sem_o1):
    cid = lax.axis_index("c")
    sid = lax.axis_index("s")
    wid = cid * NS + sid
    b = wid // WPB
    i0 = (wid % WPB) * RPW
    r_base = b * N + i0
    iota = lax.iota(jnp.int32, L)
    cols = (iota, iota + L)
    slots = ((xb0, rb0, sem_x0, sem_o0), (xb1, rb1, sem_x1, sem_o1))

    twin0 = 448 - i0
    for k in range(K):
        pltpu.sync_copy(t_hbm.at[pl.ds(k * TLEN + twin0, TWIN)],
                        twin.at[pl.ds(k * TWIN, TWIN)])

    pltpu.make_async_copy(x_hbm.at[r_base], xb0, sem_x0).start()

    def pair_step(g, carry):
        for s in range(2):
            xb, rb, sem_x, sem_o = slots[s]
            xb_n, _, sem_x_n, _ = slots[1 - s]
            rr = 2 * g + s
            r = r_base + rr
            toff = 63 - rr

            @pl.when(rr < RPW - 1)
            def _prefetch():
                pltpu.make_async_copy(x_hbm.at[r + 1], xb_n, sem_x_n).start()

            pltpu.make_async_copy(x_hbm.at[r], xb, sem_x).wait()

            @pl.when(rr >= 2)
            def _drain():
                pltpu.make_async_copy(rb, out_hbm.at[r], sem_o).wait()

            @plsc.parallel_loop(0, N, step=L, unroll=4)
            def pos_step(p0):
                xv = xb[pl.ds(p0, L)]
                basev = xv * TWIN + (iota + (p0 + toff))
                for u in range(L):
                    row = basev[u]
                    for h in range(2):
                        rb[p0 + u, pl.ds(h * L, L)] = twin[row, pl.ds(h * L, L)]
            pltpu.make_async_copy(rb, out_hbm.at[r], sem_o).start()
        return carry

    lax.fori_loop(0, RPW // 2, pair_step, 0)
    for s in range(2):
        xb, rb, sem_x, sem_o = slots[s]
        pltpu.make_async_copy(rb, out_hbm.at[r_base + RPW - 2 + s], sem_o).wait()


_sc_lookup = pl.kernel(
    _lookup_body,
    out_type=jax.ShapeDtypeStruct((ROWS, N, D), jnp.float32),
    mesh=_mesh,
    scratch_types=[
        pltpu.VMEM((K * TWIN, D), jnp.float32),
        pltpu.VMEM((N,), jnp.int32),
        pltpu.VMEM((N,), jnp.int32),
        pltpu.VMEM((N, D), jnp.float32),
        pltpu.VMEM((N, D), jnp.float32),
        pltpu.SemaphoreType.DMA,
        pltpu.SemaphoreType.DMA,
        pltpu.SemaphoreType.DMA,
        pltpu.SemaphoreType.DMA,
    ],
    compiler_params=_params,
)


@jax.jit
def kernel(x, W_bin, W_pos):
    t = _sc_build(W_bin, W_pos)
    x2 = x.astype(jnp.int32).reshape(ROWS, N)
    return _sc_lookup(x2, t).reshape(B, N, N, D)
